# baseline (speedup 1.0000x reference)
import functools
import math

import jax
import jax.numpy as jnp
from jax.experimental import pallas as pl


_QSCALE = math.log2(math.e) / 8.0


def _prep_kernel(x_ref, wqk_ref, wv_ref, xb_ref, wt_ref):
    j = pl.program_id(0)

    @pl.when(j == 0)
    def _():
        wt_ref[...] = (wqk_ref[...] * _QSCALE).astype(jnp.bfloat16).T

    @pl.when(j == 1)
    def _():
        wt_ref[...] = wqk_ref[...].astype(jnp.bfloat16).T

    @pl.when(j == 2)
    def _():
        wt_ref[...] = wv_ref[...].astype(jnp.bfloat16).T
        xb_ref[...] = x_ref[...].astype(jnp.bfloat16)


def _qkv_kernel(xb_ref, wt_ref, b_ref, out_ref):
    acc = jnp.dot(xb_ref[...], wt_ref[...], preferred_element_type=jnp.float32)
    out_ref[...] = (acc + b_ref[0]).astype(jnp.bfloat16)


def _attn_kernel(q_ref, k_ref, v_ref, wo_ref, bo_ref, out_ref):
    hp = pl.program_id(1)
    Dh = 64
    wo = wo_ref[...].astype(jnp.bfloat16)
    contrib = None
    for i in range(2):
        q = q_ref[:, i * Dh:(i + 1) * Dh]
        k = k_ref[:, i * Dh:(i + 1) * Dh]
        v = v_ref[:, i * Dh:(i + 1) * Dh]
        dots = jax.lax.dot_general(
            q, k, (((1,), (1,)), ((), ())), preferred_element_type=jnp.float32
        )
        m = jnp.max(dots, axis=-1, keepdims=True)
        e = jnp.exp2(dots - m).astype(jnp.bfloat16)
        s = jnp.sum(e, axis=-1, keepdims=True, dtype=jnp.float32)
        o = jnp.dot(e, v, preferred_element_type=jnp.float32) / s
        c = jax.lax.dot_general(
            o.astype(jnp.bfloat16), wo[:, i * Dh:(i + 1) * Dh],
            (((1,), (1,)), ((), ())), preferred_element_type=jnp.float32,
        )
        contrib = c if contrib is None else contrib + c

    @pl.when(hp == 0)
    def _():
        out_ref[...] = contrib + bo_ref[...]

    @pl.when(hp != 0)
    def _():
        out_ref[...] += contrib


@functools.partial(jax.jit, static_argnames=())
def kernel(x, Wqk, bqk, Wv, bv, Wo, bo, rot):
    del rot
    B, L, D = x.shape
    H = 16
    Dh = D // H
    x2 = x.reshape(L, D)

    ball = jnp.concatenate([bqk.at[:D].multiply(_QSCALE), bv]).reshape(3, 1, D)
    xb, Wt = pl.pallas_call(
        _prep_kernel,
        grid=(3,),
        in_specs=[
            pl.BlockSpec((L, D), lambda j: (0, 0)),
            pl.BlockSpec((D, D), lambda j: (jnp.minimum(j, 1), 0)),
            pl.BlockSpec((D, D), lambda j: (0, 0)),
        ],
        out_specs=[
            pl.BlockSpec((L, D), lambda j: (0, 0)),
            pl.BlockSpec((D, D), lambda j: (0, j)),
        ],
        out_shape=[
            jax.ShapeDtypeStruct((L, D), jnp.bfloat16),
            jax.ShapeDtypeStruct((D, 3 * D), jnp.bfloat16),
        ],
    )(x2, Wqk, Wv)
    qkv = pl.pallas_call(
        _qkv_kernel,
        grid=(3,),
        in_specs=[
            pl.BlockSpec((L, D), lambda j: (0, 0)),
            pl.BlockSpec((D, D), lambda j: (0, j)),
            pl.BlockSpec((1, 1, D), lambda j: (j, 0, 0)),
        ],
        out_specs=pl.BlockSpec((L, D), lambda j: (0, j)),
        out_shape=jax.ShapeDtypeStruct((L, 3 * D), jnp.bfloat16),
    )(xb, Wt, ball)

    HP = H // 2
    bo2 = bo.reshape(1, D)
    QB = L
    out = pl.pallas_call(
        _attn_kernel,
        grid=(L // QB, HP),
        in_specs=[
            pl.BlockSpec((QB, 2 * Dh), lambda qb, hp: (qb, hp)),
            pl.BlockSpec((L, 2 * Dh), lambda qb, hp: (0, HP + hp)),
            pl.BlockSpec((L, 2 * Dh), lambda qb, hp: (0, 2 * HP + hp)),
            pl.BlockSpec((D, 2 * Dh), lambda qb, hp: (0, hp)),
            pl.BlockSpec((1, D), lambda qb, hp: (0, 0)),
        ],
        out_specs=pl.BlockSpec((QB, D), lambda qb, hp: (qb, 0)),
        out_shape=jax.ShapeDtypeStruct((L, D), jnp.float32),
    )(qkv, qkv, qkv, Wo, bo2)

    return out.reshape(B, L, D)


# speedup vs baseline: 1.1229x; 1.1229x over previous
import functools
import math

import jax
import jax.numpy as jnp
from jax.experimental import pallas as pl


_QSCALE = math.log2(math.e) / 8.0


def _qkv_kernel(x_ref, wqk_ref, wv_ref, b_ref, out_ref):
    j = pl.program_id(0)
    xb = x_ref[...].astype(jnp.bfloat16)

    def proj(w):
        acc = jax.lax.dot_general(
            xb, w, (((1,), (1,)), ((), ())), preferred_element_type=jnp.float32
        )
        out_ref[...] = (acc + b_ref[0]).astype(jnp.bfloat16)

    @pl.when(j == 0)
    def _():
        proj((wqk_ref[...] * _QSCALE).astype(jnp.bfloat16))

    @pl.when(j == 1)
    def _():
        proj(wqk_ref[...].astype(jnp.bfloat16))

    @pl.when(j == 2)
    def _():
        proj(wv_ref[...].astype(jnp.bfloat16))


def _attn_kernel(q_ref, k_ref, v_ref, wo_ref, bo_ref, out_ref):
    hp = pl.program_id(1)
    Dh = 64
    wo = wo_ref[...].astype(jnp.bfloat16)
    contrib = None
    for i in range(2):
        q = q_ref[:, i * Dh:(i + 1) * Dh]
        k = k_ref[:, i * Dh:(i + 1) * Dh]
        v = v_ref[:, i * Dh:(i + 1) * Dh]
        dots = jax.lax.dot_general(
            q, k, (((1,), (1,)), ((), ())), preferred_element_type=jnp.float32
        )
        m = jnp.max(dots, axis=-1, keepdims=True)
        e = jnp.exp2(dots - m).astype(jnp.bfloat16)
        v_ext = jnp.concatenate(
            [v, jnp.ones((v.shape[0], 64), jnp.bfloat16)], axis=1
        )
        o_ext = jnp.dot(e, v_ext, preferred_element_type=jnp.float32)
        o = o_ext[:, :Dh] / o_ext[:, Dh:Dh + 1]
        c = jax.lax.dot_general(
            o.astype(jnp.bfloat16), wo[:, i * Dh:(i + 1) * Dh],
            (((1,), (1,)), ((), ())), preferred_element_type=jnp.float32,
        )
        contrib = c if contrib is None else contrib + c

    @pl.when(hp == 0)
    def _():
        out_ref[...] = contrib + bo_ref[...]

    @pl.when(hp != 0)
    def _():
        out_ref[...] += contrib


@functools.partial(jax.jit, static_argnames=())
def kernel(x, Wqk, bqk, Wv, bv, Wo, bo, rot):
    del rot
    B, L, D = x.shape
    H = 16
    Dh = D // H
    x2 = x.reshape(L, D)

    ball = jnp.concatenate([bqk.at[:D].multiply(_QSCALE), bv]).reshape(3, 1, D)
    qkv = pl.pallas_call(
        _qkv_kernel,
        grid=(3,),
        in_specs=[
            pl.BlockSpec((L, D), lambda j: (0, 0)),
            pl.BlockSpec((D, D), lambda j: (jnp.minimum(j, 1), 0)),
            pl.BlockSpec((D, D), lambda j: (0, 0)),
            pl.BlockSpec((1, 1, D), lambda j: (j, 0, 0)),
        ],
        out_specs=pl.BlockSpec((L, D), lambda j: (0, j)),
        out_shape=jax.ShapeDtypeStruct((L, 3 * D), jnp.bfloat16),
    )(x2, Wqk, Wv, ball)

    HP = H // 2
    bo2 = bo.reshape(1, D)
    QB = L
    out = pl.pallas_call(
        _attn_kernel,
        grid=(L // QB, HP),
        in_specs=[
            pl.BlockSpec((QB, 2 * Dh), lambda qb, hp: (qb, hp)),
            pl.BlockSpec((L, 2 * Dh), lambda qb, hp: (0, HP + hp)),
            pl.BlockSpec((L, 2 * Dh), lambda qb, hp: (0, 2 * HP + hp)),
            pl.BlockSpec((D, 2 * Dh), lambda qb, hp: (0, hp)),
            pl.BlockSpec((1, D), lambda qb, hp: (0, 0)),
        ],
        out_specs=pl.BlockSpec((QB, D), lambda qb, hp: (qb, 0)),
        out_shape=jax.ShapeDtypeStruct((L, D), jnp.float32),
    )(qkv, qkv, qkv, Wo, bo2)

    return out.reshape(B, L, D)
